# Initial kernel scaffold; baseline (speedup 1.0000x reference)
import jax
import jax.numpy as jnp
from jax.experimental import pallas as pl


def kernel(aa_indices, mod_x, emb_table, W):
    raise NotImplementedError("write your pallas kernel here")


# speedup vs baseline: 3.3793x; 3.3793x over previous
import numpy as np
import jax
import jax.numpy as jnp
from jax import lax
from jax.experimental import pallas as pl

_MOD_F = 109
_EMB_D = 120
_OUT_F = 128
_MAX_LEN = 200
_FIRST_K = 6
_VOCAB_PAD = 32


def _pe_const():
    position = np.arange(_MAX_LEN, dtype=np.float32)[:, None]
    div_term = np.exp(
        np.arange(0, _OUT_F, 2, dtype=np.float32) * (-np.log(_MAX_LEN) / _OUT_F)
    )
    pe = np.zeros((_MAX_LEN, _OUT_F), dtype=np.float32)
    pe[:, 0::2] = np.sin(position * div_term)
    pe[:, 1::2] = np.cos(position * div_term)
    return pe


_PE = _pe_const()


def _body(idx_ref, mod_ref, e_ref, w_ref, pe_ref, out_ref):
    bb, seq, nf = mod_ref.shape
    t = bb * seq
    m = mod_ref[...].reshape(t, nf)
    mod_part = lax.dot_general(
        m, w_ref[...], (((1,), (0,)), ((), ())),
        preferred_element_type=jnp.float32,
        precision=lax.Precision.HIGHEST,
    )
    idx3 = idx_ref[...][:, :, None]
    onehot = (idx3 == lax.broadcasted_iota(jnp.int32, (bb, seq, _VOCAB_PAD), 2)
              ).astype(jnp.float32).reshape(t, _VOCAB_PAD)
    emb_part = lax.dot_general(
        onehot, e_ref[...], (((1,), (0,)), ((), ())),
        preferred_element_type=jnp.float32,
        precision=lax.Precision.HIGHEST,
    )
    x = (mod_part + emb_part).reshape(bb, seq, _OUT_F)
    out_ref[...] = x + pe_ref[...][None, :, :]


def kernel(aa_indices, mod_x, emb_table, W):
    B, S, F = mod_x.shape
    vocab = emb_table.shape[0]
    eye = jnp.zeros((F, _OUT_F), jnp.float32)
    eye = eye.at[jnp.arange(_FIRST_K), _EMB_D + jnp.arange(_FIRST_K)].set(1.0)
    w_full = eye.at[_FIRST_K:, _EMB_D + _FIRST_K:].set(W.T)
    e_full = jnp.zeros((_VOCAB_PAD, _OUT_F), jnp.float32)
    e_full = e_full.at[:vocab, :_EMB_D].set(emb_table)
    pe = jnp.asarray(_PE[:S])

    BB = 8
    out = pl.pallas_call(
        _body,
        grid=(B // BB,),
        in_specs=[
            pl.BlockSpec((BB, S), lambda i: (i, 0)),
            pl.BlockSpec((BB, S, F), lambda i: (i, 0, 0)),
            pl.BlockSpec((_VOCAB_PAD, _OUT_F), lambda i: (0, 0)),
            pl.BlockSpec((F, _OUT_F), lambda i: (0, 0)),
            pl.BlockSpec((S, _OUT_F), lambda i: (0, 0)),
        ],
        out_specs=pl.BlockSpec((BB, S, _OUT_F), lambda i: (i, 0, 0)),
        out_shape=jax.ShapeDtypeStruct((B, S, _OUT_F), jnp.float32),
    )(aa_indices, mod_x, e_full, w_full, pe)
    return out


# speedup vs baseline: 4.3934x; 1.3001x over previous
import numpy as np
import jax
import jax.numpy as jnp
from jax import lax
from jax.experimental import pallas as pl

_MOD_F = 109
_EMB_D = 120
_OUT_F = 128
_MAX_LEN = 200
_FIRST_K = 6
_VOCAB_PAD = 32


def _pe_const():
    position = np.arange(_MAX_LEN, dtype=np.float32)[:, None]
    div_term = np.exp(
        np.arange(0, _OUT_F, 2, dtype=np.float32) * (-np.log(_MAX_LEN) / _OUT_F)
    )
    pe = np.zeros((_MAX_LEN, _OUT_F), dtype=np.float32)
    pe[:, 0::2] = np.sin(position * div_term)
    pe[:, 1::2] = np.cos(position * div_term)
    return pe


_PE = _pe_const()


def _body(idx_ref, mod_ref, e_ref, w_ref, pe_ref, out_ref):
    bb, seq, nf = mod_ref.shape
    t = bb * seq
    m = mod_ref[...].reshape(t, nf)
    mod_part = lax.dot_general(
        m, w_ref[...], (((1,), (0,)), ((), ())),
        preferred_element_type=jnp.float32,
        precision=lax.Precision.DEFAULT,
    )
    idx3 = idx_ref[...][:, :, None]
    onehot = (idx3 == lax.broadcasted_iota(jnp.int32, (bb, seq, _VOCAB_PAD), 2)
              ).astype(jnp.float32).reshape(t, _VOCAB_PAD)
    emb_part = lax.dot_general(
        onehot, e_ref[...], (((1,), (0,)), ((), ())),
        preferred_element_type=jnp.float32,
        precision=lax.Precision.DEFAULT,
    )
    x = (mod_part + emb_part).reshape(bb, seq, _OUT_F)
    out_ref[...] = x + pe_ref[...][None, :, :]


def kernel(aa_indices, mod_x, emb_table, W):
    B, S, F = mod_x.shape
    vocab = emb_table.shape[0]
    eye = jnp.zeros((F, _OUT_F), jnp.float32)
    eye = eye.at[jnp.arange(_FIRST_K), _EMB_D + jnp.arange(_FIRST_K)].set(1.0)
    w_full = eye.at[_FIRST_K:, _EMB_D + _FIRST_K:].set(W.T)
    e_full = jnp.zeros((_VOCAB_PAD, _OUT_F), jnp.float32)
    e_full = e_full.at[:vocab, :_EMB_D].set(emb_table)
    pe = jnp.asarray(_PE[:S])

    BB = 8
    out = pl.pallas_call(
        _body,
        grid=(B // BB,),
        in_specs=[
            pl.BlockSpec((BB, S), lambda i: (i, 0)),
            pl.BlockSpec((BB, S, F), lambda i: (i, 0, 0)),
            pl.BlockSpec((_VOCAB_PAD, _OUT_F), lambda i: (0, 0)),
            pl.BlockSpec((F, _OUT_F), lambda i: (0, 0)),
            pl.BlockSpec((S, _OUT_F), lambda i: (0, 0)),
        ],
        out_specs=pl.BlockSpec((BB, S, _OUT_F), lambda i: (i, 0, 0)),
        out_shape=jax.ShapeDtypeStruct((B, S, _OUT_F), jnp.float32),
    )(aa_indices, mod_x, e_full, w_full, pe)
    return out


# speedup vs baseline: 5.2401x; 1.1927x over previous
import numpy as np
import jax
import jax.numpy as jnp
from jax import lax
from jax.experimental import pallas as pl

_MOD_F = 109
_EMB_D = 120
_OUT_F = 128
_MAX_LEN = 200
_FIRST_K = 6
_VOCAB_PAD = 32


def _pe_const():
    position = np.arange(_MAX_LEN, dtype=np.float32)[:, None]
    div_term = np.exp(
        np.arange(0, _OUT_F, 2, dtype=np.float32) * (-np.log(_MAX_LEN) / _OUT_F)
    )
    pe = np.zeros((_MAX_LEN, _OUT_F), dtype=np.float32)
    pe[:, 0::2] = np.sin(position * div_term)
    pe[:, 1::2] = np.cos(position * div_term)
    return pe


_PE = _pe_const()


def _body(idx_ref, mod_ref, e_ref, w_ref, pe_ref, out_ref):
    bb, seq, nf = mod_ref.shape
    t = bb * seq
    m = mod_ref[...].reshape(t, nf)
    mod_part = lax.dot_general(
        m, w_ref[...], (((1,), (0,)), ((), ())),
        preferred_element_type=jnp.float32,
        precision=lax.Precision.DEFAULT,
    )
    idx3 = idx_ref[...][:, :, None]
    onehot = (idx3 == lax.broadcasted_iota(jnp.int32, (bb, seq, _VOCAB_PAD), 2)
              ).astype(jnp.float32).reshape(t, _VOCAB_PAD)
    emb_part = lax.dot_general(
        onehot, e_ref[...], (((1,), (0,)), ((), ())),
        preferred_element_type=jnp.float32,
        precision=lax.Precision.DEFAULT,
    )
    x = (mod_part + emb_part).reshape(bb, seq, _OUT_F)
    out_ref[...] = x + pe_ref[...][None, :, :]


def kernel(aa_indices, mod_x, emb_table, W):
    B, S, F = mod_x.shape
    vocab = emb_table.shape[0]
    eye = jnp.zeros((F, _OUT_F), jnp.float32)
    eye = eye.at[jnp.arange(_FIRST_K), _EMB_D + jnp.arange(_FIRST_K)].set(1.0)
    w_full = eye.at[_FIRST_K:, _EMB_D + _FIRST_K:].set(W.T)
    e_full = jnp.zeros((_VOCAB_PAD, _OUT_F), jnp.float32)
    e_full = e_full.at[:vocab, :_EMB_D].set(emb_table)
    pe = jnp.asarray(_PE[:S])

    BB = 16
    out = pl.pallas_call(
        _body,
        grid=(B // BB,),
        in_specs=[
            pl.BlockSpec((BB, S), lambda i: (i, 0)),
            pl.BlockSpec((BB, S, F), lambda i: (i, 0, 0)),
            pl.BlockSpec((_VOCAB_PAD, _OUT_F), lambda i: (0, 0)),
            pl.BlockSpec((F, _OUT_F), lambda i: (0, 0)),
            pl.BlockSpec((S, _OUT_F), lambda i: (0, 0)),
        ],
        out_specs=pl.BlockSpec((BB, S, _OUT_F), lambda i: (i, 0, 0)),
        out_shape=jax.ShapeDtypeStruct((B, S, _OUT_F), jnp.float32),
    )(aa_indices, mod_x, e_full, w_full, pe)
    return out


# speedup vs baseline: 5.8214x; 1.1109x over previous
import numpy as np
import jax
import jax.numpy as jnp
from jax import lax
from jax.experimental import pallas as pl

_MOD_F = 109
_EMB_D = 120
_OUT_F = 128
_MAX_LEN = 200
_FIRST_K = 6
_VOCAB_PAD = 32


def _pe_const():
    position = np.arange(_MAX_LEN, dtype=np.float32)[:, None]
    div_term = np.exp(
        np.arange(0, _OUT_F, 2, dtype=np.float32) * (-np.log(_MAX_LEN) / _OUT_F)
    )
    pe = np.zeros((_MAX_LEN, _OUT_F), dtype=np.float32)
    pe[:, 0::2] = np.sin(position * div_term)
    pe[:, 1::2] = np.cos(position * div_term)
    return pe


_PE = _pe_const()


def _body(idx_ref, mod_ref, e_ref, w_ref, pe_ref, out_ref):
    bb, seq, nf = mod_ref.shape
    t = bb * seq
    m = mod_ref[...].reshape(t, nf)
    mod_part = lax.dot_general(
        m, w_ref[...], (((1,), (0,)), ((), ())),
        preferred_element_type=jnp.float32,
        precision=lax.Precision.DEFAULT,
    )
    idx3 = idx_ref[...][:, :, None]
    onehot = (idx3 == lax.broadcasted_iota(jnp.int32, (bb, seq, _VOCAB_PAD), 2)
              ).astype(jnp.float32).reshape(t, _VOCAB_PAD)
    emb_part = lax.dot_general(
        onehot, e_ref[...], (((1,), (0,)), ((), ())),
        preferred_element_type=jnp.float32,
        precision=lax.Precision.DEFAULT,
    )
    x = (mod_part + emb_part).reshape(bb, seq, _OUT_F)
    out_ref[...] = x + pe_ref[...][None, :, :]


def kernel(aa_indices, mod_x, emb_table, W):
    B, S, F = mod_x.shape
    vocab = emb_table.shape[0]
    eye = jnp.zeros((F, _OUT_F), jnp.float32)
    eye = eye.at[jnp.arange(_FIRST_K), _EMB_D + jnp.arange(_FIRST_K)].set(1.0)
    w_full = eye.at[_FIRST_K:, _EMB_D + _FIRST_K:].set(W.T)
    e_full = jnp.zeros((_VOCAB_PAD, _OUT_F), jnp.float32)
    e_full = e_full.at[:vocab, :_EMB_D].set(emb_table)
    pe = jnp.asarray(_PE[:S])

    BB = 32
    out = pl.pallas_call(
        _body,
        grid=(B // BB,),
        in_specs=[
            pl.BlockSpec((BB, S), lambda i: (i, 0)),
            pl.BlockSpec((BB, S, F), lambda i: (i, 0, 0)),
            pl.BlockSpec((_VOCAB_PAD, _OUT_F), lambda i: (0, 0)),
            pl.BlockSpec((F, _OUT_F), lambda i: (0, 0)),
            pl.BlockSpec((S, _OUT_F), lambda i: (0, 0)),
        ],
        out_specs=pl.BlockSpec((BB, S, _OUT_F), lambda i: (i, 0, 0)),
        out_shape=jax.ShapeDtypeStruct((B, S, _OUT_F), jnp.float32),
    )(aa_indices, mod_x, e_full, w_full, pe)
    return out


# speedup vs baseline: 6.0303x; 1.0359x over previous
import numpy as np
import jax
import jax.numpy as jnp
from jax import lax
from jax.experimental import pallas as pl

_MOD_F = 109
_EMB_D = 120
_OUT_F = 128
_MAX_LEN = 200
_FIRST_K = 6
_VOCAB_PAD = 32


def _pe_const():
    position = np.arange(_MAX_LEN, dtype=np.float32)[:, None]
    div_term = np.exp(
        np.arange(0, _OUT_F, 2, dtype=np.float32) * (-np.log(_MAX_LEN) / _OUT_F)
    )
    pe = np.zeros((_MAX_LEN, _OUT_F), dtype=np.float32)
    pe[:, 0::2] = np.sin(position * div_term)
    pe[:, 1::2] = np.cos(position * div_term)
    return pe


_PE = _pe_const()


def _body(idx_ref, mod_ref, e_ref, w_ref, pe_ref, out_ref):
    bb, seq, nf = mod_ref.shape
    t = bb * seq
    m = mod_ref[...].reshape(t, nf)
    mod_part = lax.dot_general(
        m, w_ref[...], (((1,), (0,)), ((), ())),
        preferred_element_type=jnp.float32,
        precision=lax.Precision.DEFAULT,
    )
    idx3 = idx_ref[...][:, :, None]
    onehot = (idx3 == lax.broadcasted_iota(jnp.int32, (bb, seq, _VOCAB_PAD), 2)
              ).astype(jnp.float32).reshape(t, _VOCAB_PAD)
    emb_part = lax.dot_general(
        onehot, e_ref[...], (((1,), (0,)), ((), ())),
        preferred_element_type=jnp.float32,
        precision=lax.Precision.DEFAULT,
    )
    x = (mod_part + emb_part).reshape(bb, seq, _OUT_F)
    out_ref[...] = x + pe_ref[...][None, :, :]


def kernel(aa_indices, mod_x, emb_table, W):
    B, S, F = mod_x.shape
    vocab = emb_table.shape[0]
    eye = jnp.zeros((F, _OUT_F), jnp.float32)
    eye = eye.at[jnp.arange(_FIRST_K), _EMB_D + jnp.arange(_FIRST_K)].set(1.0)
    w_full = eye.at[_FIRST_K:, _EMB_D + _FIRST_K:].set(W.T)
    e_full = jnp.zeros((_VOCAB_PAD, _OUT_F), jnp.float32)
    e_full = e_full.at[:vocab, :_EMB_D].set(emb_table)
    pe = jnp.asarray(_PE[:S])

    BB = 64
    out = pl.pallas_call(
        _body,
        grid=(B // BB,),
        in_specs=[
            pl.BlockSpec((BB, S), lambda i: (i, 0)),
            pl.BlockSpec((BB, S, F), lambda i: (i, 0, 0)),
            pl.BlockSpec((_VOCAB_PAD, _OUT_F), lambda i: (0, 0)),
            pl.BlockSpec((F, _OUT_F), lambda i: (0, 0)),
            pl.BlockSpec((S, _OUT_F), lambda i: (0, 0)),
        ],
        out_specs=pl.BlockSpec((BB, S, _OUT_F), lambda i: (i, 0, 0)),
        out_shape=jax.ShapeDtypeStruct((B, S, _OUT_F), jnp.float32),
    )(aa_indices, mod_x, e_full, w_full, pe)
    return out


# speedup vs baseline: 6.4275x; 1.0659x over previous
import numpy as np
import jax
import jax.numpy as jnp
from jax import lax
from jax.experimental import pallas as pl
from jax.experimental.pallas import tpu as pltpu

_MOD_F = 109
_EMB_D = 120
_OUT_F = 128
_MAX_LEN = 200
_FIRST_K = 6
_VOCAB_PAD = 32

_NCH = 32
_DEPTH = 6
_ODEPTH = 4


def _pe_const():
    position = np.arange(_MAX_LEN, dtype=np.float32)[:, None]
    div_term = np.exp(
        np.arange(0, _OUT_F, 2, dtype=np.float32) * (-np.log(_MAX_LEN) / _OUT_F)
    )
    pe = np.zeros((_MAX_LEN, _OUT_F), dtype=np.float32)
    pe[:, 0::2] = np.sin(position * div_term)
    pe[:, 1::2] = np.cos(position * div_term)
    return pe


_PE = _pe_const()


def _make_body(rows):
    def _body(idx_hbm, mod_hbm, e_ref, w_ref, pe_ref, out_hbm,
              ibuf, obuf, xbuf, isem, osem, xsem):
        in_cp = [pltpu.make_async_copy(mod_hbm.at[pl.ds(i * rows, rows)],
                                       ibuf.at[i % _DEPTH], isem.at[i % _DEPTH])
                 for i in range(_NCH)]
        ix_cp = [pltpu.make_async_copy(idx_hbm.at[i],
                                       xbuf.at[i % _DEPTH], xsem.at[i % _DEPTH])
                 for i in range(_NCH)]
        out_cp = [pltpu.make_async_copy(obuf.at[i % _ODEPTH],
                                        out_hbm.at[pl.ds(i * rows, rows)],
                                        osem.at[i % _ODEPTH])
                  for i in range(_NCH)]
        for j in range(_DEPTH):
            in_cp[j].start()
            ix_cp[j].start()
        for i in range(_NCH):
            sl = i % _DEPTH
            in_cp[i].wait()
            ix_cp[i].wait()
            if i >= _ODEPTH:
                out_cp[i - _ODEPTH].wait()
            oh = (xbuf[sl][:, :, None]
                  == lax.broadcasted_iota(jnp.int32, (8, rows // 8, _VOCAB_PAD), 2)
                  ).astype(jnp.float32).reshape(rows, _VOCAB_PAD)
            emb_part = lax.dot_general(
                oh, e_ref[...], (((1,), (0,)), ((), ())),
                preferred_element_type=jnp.float32)
            mod_part = lax.dot_general(
                ibuf[sl], w_ref[...], (((1,), (0,)), ((), ())),
                preferred_element_type=jnp.float32)
            obuf[i % _ODEPTH] = emb_part + mod_part + pe_ref[...]
            out_cp[i].start()
            if i + _DEPTH < _NCH:
                in_cp[i + _DEPTH].start()
                ix_cp[i + _DEPTH].start()
        for i in range(_NCH - _ODEPTH, _NCH):
            out_cp[i].wait()
    return _body


def kernel(aa_indices, mod_x, emb_table, W):
    B, S, F = mod_x.shape
    vocab = emb_table.shape[0]
    T = B * S
    rows = T // _NCH
    sub = rows // S
    eye = jnp.zeros((F, _OUT_F), jnp.float32)
    eye = eye.at[jnp.arange(_FIRST_K), _EMB_D + jnp.arange(_FIRST_K)].set(1.0)
    w_full = eye.at[_FIRST_K:, _EMB_D + _FIRST_K:].set(W.T)
    e_full = jnp.zeros((_VOCAB_PAD, _OUT_F), jnp.float32)
    e_full = e_full.at[:vocab, :_EMB_D].set(emb_table)
    pe_t = jnp.tile(jnp.asarray(_PE[:S]), (sub, 1))

    mod2 = mod_x.reshape(T, F)
    idx3 = aa_indices.reshape(_NCH, 8, rows // 8).astype(jnp.int32)

    out = pl.pallas_call(
        _make_body(rows),
        in_specs=[
            pl.BlockSpec(memory_space=pltpu.HBM),
            pl.BlockSpec(memory_space=pltpu.HBM),
            pl.BlockSpec(memory_space=pltpu.VMEM),
            pl.BlockSpec(memory_space=pltpu.VMEM),
            pl.BlockSpec(memory_space=pltpu.VMEM),
        ],
        out_specs=pl.BlockSpec(memory_space=pltpu.HBM),
        out_shape=jax.ShapeDtypeStruct((T, _OUT_F), jnp.float32),
        scratch_shapes=[
            pltpu.VMEM((_DEPTH, rows, F), jnp.float32),
            pltpu.VMEM((_ODEPTH, rows, _OUT_F), jnp.float32),
            pltpu.VMEM((_DEPTH, 8, rows // 8), jnp.int32),
            pltpu.SemaphoreType.DMA((_DEPTH,)),
            pltpu.SemaphoreType.DMA((_ODEPTH,)),
            pltpu.SemaphoreType.DMA((_DEPTH,)),
        ],
        compiler_params=pltpu.CompilerParams(
            vmem_limit_bytes=100 * 1024 * 1024,
        ),
    )(idx3, mod2, e_full, w_full, pe_t)
    return out.reshape(B, S, _OUT_F)


# speedup vs baseline: 6.4637x; 1.0056x over previous
import numpy as np
import jax
import jax.numpy as jnp
from jax import lax
from jax.experimental import pallas as pl
from jax.experimental.pallas import tpu as pltpu

_MOD_F = 109
_EMB_D = 120
_OUT_F = 128
_MAX_LEN = 200
_FIRST_K = 6
_VOCAB_PAD = 32

_NCH = 64
_DEPTH = 10
_ODEPTH = 4


def _pe_const():
    position = np.arange(_MAX_LEN, dtype=np.float32)[:, None]
    div_term = np.exp(
        np.arange(0, _OUT_F, 2, dtype=np.float32) * (-np.log(_MAX_LEN) / _OUT_F)
    )
    pe = np.zeros((_MAX_LEN, _OUT_F), dtype=np.float32)
    pe[:, 0::2] = np.sin(position * div_term)
    pe[:, 1::2] = np.cos(position * div_term)
    return pe


_PE = _pe_const()


def _make_body(rows):
    def _body(idx_hbm, mod_hbm, e_ref, w_ref, pe_ref, out_hbm,
              ibuf, obuf, xbuf, isem, osem, xsem):
        in_cp = [pltpu.make_async_copy(mod_hbm.at[pl.ds(i * rows, rows)],
                                       ibuf.at[i % _DEPTH], isem.at[i % _DEPTH])
                 for i in range(_NCH)]
        ix_cp = [pltpu.make_async_copy(idx_hbm.at[i],
                                       xbuf.at[i % _DEPTH], xsem.at[i % _DEPTH])
                 for i in range(_NCH)]
        out_cp = [pltpu.make_async_copy(obuf.at[i % _ODEPTH],
                                        out_hbm.at[pl.ds(i * rows, rows)],
                                        osem.at[i % _ODEPTH])
                  for i in range(_NCH)]
        for j in range(_DEPTH):
            in_cp[j].start()
            ix_cp[j].start()
        for i in range(_NCH):
            sl = i % _DEPTH
            in_cp[i].wait()
            ix_cp[i].wait()
            if i >= _ODEPTH:
                out_cp[i - _ODEPTH].wait()
            oh = (xbuf[sl][:, :, None]
                  == lax.broadcasted_iota(jnp.int32, (8, rows // 8, _VOCAB_PAD), 2)
                  ).astype(jnp.float32).reshape(rows, _VOCAB_PAD)
            emb_part = lax.dot_general(
                oh, e_ref[...], (((1,), (0,)), ((), ())),
                preferred_element_type=jnp.float32)
            mod_part = lax.dot_general(
                ibuf[sl], w_ref[...], (((1,), (0,)), ((), ())),
                preferred_element_type=jnp.float32)
            obuf[i % _ODEPTH] = emb_part + mod_part + pe_ref[...]
            out_cp[i].start()
            if i + _DEPTH < _NCH:
                in_cp[i + _DEPTH].start()
                ix_cp[i + _DEPTH].start()
        for i in range(_NCH - _ODEPTH, _NCH):
            out_cp[i].wait()
    return _body


def kernel(aa_indices, mod_x, emb_table, W):
    B, S, F = mod_x.shape
    vocab = emb_table.shape[0]
    T = B * S
    rows = T // _NCH
    sub = rows // S
    eye = jnp.zeros((F, _OUT_F), jnp.float32)
    eye = eye.at[jnp.arange(_FIRST_K), _EMB_D + jnp.arange(_FIRST_K)].set(1.0)
    w_full = eye.at[_FIRST_K:, _EMB_D + _FIRST_K:].set(W.T)
    e_full = jnp.zeros((_VOCAB_PAD, _OUT_F), jnp.float32)
    e_full = e_full.at[:vocab, :_EMB_D].set(emb_table)
    pe_t = jnp.tile(jnp.asarray(_PE[:S]), (sub, 1))

    mod2 = mod_x.reshape(T, F)
    idx3 = aa_indices.reshape(_NCH, 8, rows // 8).astype(jnp.int32)

    out = pl.pallas_call(
        _make_body(rows),
        in_specs=[
            pl.BlockSpec(memory_space=pltpu.HBM),
            pl.BlockSpec(memory_space=pltpu.HBM),
            pl.BlockSpec(memory_space=pltpu.VMEM),
            pl.BlockSpec(memory_space=pltpu.VMEM),
            pl.BlockSpec(memory_space=pltpu.VMEM),
        ],
        out_specs=pl.BlockSpec(memory_space=pltpu.HBM),
        out_shape=jax.ShapeDtypeStruct((T, _OUT_F), jnp.float32),
        scratch_shapes=[
            pltpu.VMEM((_DEPTH, rows, F), jnp.float32),
            pltpu.VMEM((_ODEPTH, rows, _OUT_F), jnp.float32),
            pltpu.VMEM((_DEPTH, 8, rows // 8), jnp.int32),
            pltpu.SemaphoreType.DMA((_DEPTH,)),
            pltpu.SemaphoreType.DMA((_ODEPTH,)),
            pltpu.SemaphoreType.DMA((_DEPTH,)),
        ],
        compiler_params=pltpu.CompilerParams(
            vmem_limit_bytes=100 * 1024 * 1024,
        ),
    )(idx3, mod2, e_full, w_full, pe_t)
    return out.reshape(B, S, _OUT_F)
